# Initial kernel scaffold; baseline (speedup 1.0000x reference)
import jax
import jax.numpy as jnp
from jax.experimental import pallas as pl


def kernel(feat_a, feat_b, feat_c):
    raise NotImplementedError("write your pallas kernel here")


# speedup vs baseline: 1.0393x; 1.0393x over previous
import functools

import jax
import jax.numpy as jnp
from jax import lax
from jax.experimental import pallas as pl
from jax.experimental.pallas import tpu as pltpu
from jax.experimental.pallas import tpu_sc as plsc

NC = 2
NS = 16
NW = NC * NS
L = 16

BETA = 0.1
NBINS = 256
CHUNK = 8192

SIZES = (16 * 768 * 24 * 24, 16 * 384 * 48 * 48, 16 * 192 * 96 * 96)

_mesh = plsc.VectorSubcoreMesh(
    core_axis_name="c", subcore_axis_name="s", num_cores=NC, num_subcores=NS
)


def _wid():
    return lax.axis_index("s") * NC + lax.axis_index("c")


def _pass1_body(fa, fb, fc, out, buf, stage):
    wid = _wid()
    feats = (fa, fb, fc)
    for t in range(3):
        n_per = SIZES[t] // NW
        nchunks = n_per // CHUNK
        base = wid * n_per

        def chunk_step(ci, carry, f=feats[t], base=base):
            mn, mx, sa = carry
            pltpu.sync_copy(f.at[pl.ds(base + ci * CHUNK, CHUNK)], buf)

            def vec_step(i, c):
                m0, m1, s0 = c
                v = buf[pl.ds(i * L, L)]
                return (jnp.minimum(m0, v), jnp.maximum(m1, v),
                        s0 + jnp.abs(v))

            return lax.fori_loop(0, CHUNK // L, vec_step, (mn, mx, sa))

        init = (
            jnp.full((L,), jnp.inf, jnp.float32),
            jnp.full((L,), -jnp.inf, jnp.float32),
            jnp.zeros((L,), jnp.float32),
        )
        mn, mx, sa = lax.fori_loop(0, nchunks, chunk_step, init)
        stage[pl.ds(0, L)] = mn
        stage[pl.ds(L, L)] = mx
        stage[pl.ds(2 * L, L)] = sa
        pltpu.sync_copy(stage, out.at[t, wid])


def _pass2_body(fa, fb, fc, parts, out, buf, hist, pv, hstage):
    wid = _wid()
    pltpu.sync_copy(parts, pv)
    lane_addr = lax.iota(jnp.int32, L) * NBINS
    ones = jnp.ones((L,), jnp.float32)
    zeros = jnp.zeros((L,), jnp.float32)
    feats = (fa, fb, fc)

    for t in range(3):
        n_per = SIZES[t] // NW
        nchunks = n_per // CHUNK
        base = wid * n_per

        def red_step(w, c, t=t):
            m0, m1 = c
            off = (t * NW + w) * 3 * L
            return (jnp.minimum(m0, pv[pl.ds(off, L)]),
                    jnp.maximum(m1, pv[pl.ds(off + L, L)]))

        mnv, mxv = lax.fori_loop(
            0, NW, red_step,
            (jnp.full((L,), jnp.inf, jnp.float32),
             jnp.full((L,), -jnp.inf, jnp.float32)))
        gmin, gmax = mnv[0], mxv[0]
        for j in range(1, L):
            gmin = jnp.minimum(gmin, mnv[j])
            gmax = jnp.maximum(gmax, mxv[j])
        gmin_v = jnp.full((L,), gmin, jnp.float32)
        denom_v = jnp.full((L,), gmax - gmin + 1e-08, jnp.float32)
        scale_v = 256.0 / denom_v

        def zero_step(j, _):
            hist[pl.ds(j * L, L)] = zeros
            return 0

        lax.fori_loop(0, (NS * NBINS) // L, zero_step, 0)

        def chunk_step(ci, _, f=feats[t], base=base, scale=scale_v,
                       gmin=gmin_v):
            pltpu.sync_copy(f.at[pl.ds(base + ci * CHUNK, CHUNK)], buf)

            def vec_step(i, __):
                v = buf[pl.ds(i * L, L)]
                y = (v - gmin) * scale
                yi = jnp.clip(y.astype(jnp.int32), 0, NBINS - 1)
                plsc.addupdate_scatter(hist, [lane_addr + yi], ones)
                return 0

            return lax.fori_loop(0, CHUNK // L, vec_step, 0)

        lax.fori_loop(0, nchunks, chunk_step, 0)

        def fold_step(j, _):
            def lane_step(l, acc):
                return acc + hist[pl.ds(l * NBINS + j * L, L)]

            hstage[pl.ds(j * L, L)] = lax.fori_loop(0, NS, lane_step, zeros)
            return 0

        lax.fori_loop(0, NBINS // L, fold_step, 0)
        pltpu.sync_copy(hstage, out.at[t, wid])


_pass1 = functools.partial(
    pl.kernel,
    out_type=jax.ShapeDtypeStruct((3, NW, 3 * L), jnp.float32),
    mesh=_mesh,
    compiler_params=pltpu.CompilerParams(needs_layout_passes=False),
    scratch_types=[
        pltpu.VMEM((CHUNK,), jnp.float32),
        pltpu.VMEM((3 * L,), jnp.float32),
    ],
)(_pass1_body)

_pass2 = functools.partial(
    pl.kernel,
    out_type=jax.ShapeDtypeStruct((3, NW, NBINS), jnp.float32),
    mesh=_mesh,
    compiler_params=pltpu.CompilerParams(needs_layout_passes=False),
    scratch_types=[
        pltpu.VMEM((CHUNK,), jnp.float32),
        pltpu.VMEM((NS * NBINS,), jnp.float32),
        pltpu.VMEM((3 * NW * 3 * L,), jnp.float32),
        pltpu.VMEM((NBINS,), jnp.float32),
    ],
)(_pass2_body)


def _finalize_body(parts_ref, hist_ref, out_ref):
    parts = parts_ref[...]
    hists = hist_ref[...]
    h = jnp.sum(hists, axis=1)
    total = jnp.sum(h, axis=1, keepdims=True)
    p = h / total
    ent = -jnp.sum(p * jnp.log2(p + 1e-08), axis=1)
    sumabs = jnp.sum(parts[:, :, 2 * L:3 * L], axis=(1, 2))
    sparsity = (sumabs[0] / SIZES[0] + sumabs[1] / SIZES[1]
                + sumabs[2] / SIZES[2]) / 3.0
    loss = sparsity + BETA * jnp.mean(ent)
    out_ref[...] = jnp.reshape(loss, (1, 1))


def kernel(feat_a, feat_b, feat_c):
    flats = [f.reshape(-1) for f in (feat_a, feat_b, feat_c)]
    parts = _pass1(*flats)
    hists = _pass2(*flats, parts.reshape(-1))
    loss = pl.pallas_call(
        _finalize_body,
        out_shape=jax.ShapeDtypeStruct((1, 1), jnp.float32),
    )(parts, hists)
    return loss[0, 0]
